# baseline (speedup 1.0000x reference)
import functools

import jax
import jax.numpy as jnp
from jax import lax
from jax.experimental import pallas as pl
from jax.experimental.pallas import tpu as pltpu

_NJ = 4


def _partial_kernel(ve_ref, la_ref, x_ref, w1_ref, u_ref,
                    hp_ref):
    f32 = jnp.float32
    j = pl.program_id(1)
    X = x_ref[...]
    Ucj = u_ref[...]

    T2 = lax.dot_general(X, Ucj, (((0,), (0,)), ((), ())),
                         preferred_element_type=f32)

    vla = jnp.power(la_ref[...], ve_ref[0])

    Tw2 = lax.dot_general(w1_ref[...], T2 * vla, (((1,), (0,)), ((), ())),
                          preferred_element_type=f32)

    Hj = lax.dot_general(Ucj, Tw2, (((1,), (1,)), ((), ())),
                         preferred_element_type=f32)

    @pl.when(j == 0)
    def _init():
        hp_ref[0] = Hj

    @pl.when(j > 0)
    def _acc():
        hp_ref[0] = hp_ref[0] + Hj


def _combine_kernel(hp_ref, b1_ref, gamma_ref, beta_ref, w2_ref, b2_ref,
                    out_ref, hid_ref):
    f32 = jnp.float32
    c = pl.program_id(0)
    nh = hid_ref.shape[0]

    H = hp_ref[0] + hp_ref[1] + b1_ref[...]

    mean = jnp.mean(H, axis=0, keepdims=True)
    var = jnp.mean(jnp.square(H - mean), axis=0, keepdims=True)

    Hh = (hp_ref[0, pl.ds(c * nh, nh), :] + hp_ref[1, pl.ds(c * nh, nh), :]
          + b1_ref[...])
    hid_ref[...] = Hh

    Hn = (Hh - mean) * lax.rsqrt(var + 1e-5)
    Hn = Hn * gamma_ref[...] + beta_ref[...]
    Hr = jnp.maximum(Hn, 0.0)

    logits = lax.dot_general(Hr, w2_ref[...], (((1,), (1,)), ((), ())),
                             preferred_element_type=f32) + b2_ref[...]

    m = jnp.max(logits, axis=1, keepdims=True)
    z = logits - m
    lse = jnp.log(jnp.sum(jnp.exp(z), axis=1, keepdims=True))
    out_ref[...] = z - lse


def kernel(X, La, U, ve, W_w, W_b, bn_gamma, bn_beta, MLP_w, MLP_b):
    N, in_c = X.shape
    hidden = W_w.shape[0]
    out_c = MLP_w.shape[0]
    nh = N // 2
    cj = nh // _NJ

    smem = pl.BlockSpec(memory_space=pltpu.MemorySpace.SMEM)

    hp = pl.pallas_call(
        _partial_kernel,
        grid=(2, _NJ),
        out_shape=jax.ShapeDtypeStruct((2, N, hidden), jnp.float32),
        in_specs=[
            smem,
            pl.BlockSpec((1, cj), lambda c, j: (0, c * _NJ + j)),
            pl.BlockSpec((N, in_c), lambda c, j: (0, 0)),
            pl.BlockSpec((hidden, in_c), lambda c, j: (0, 0)),
            pl.BlockSpec((N, cj), lambda c, j: (0, c * _NJ + j)),
        ],
        out_specs=pl.BlockSpec((1, N, hidden), lambda c, j: (c, 0, 0)),
        compiler_params=pltpu.CompilerParams(
            dimension_semantics=("parallel", "arbitrary")),
    )(
        ve.astype(jnp.float32).reshape(1),
        La.reshape(1, N).astype(jnp.float32),
        X.astype(jnp.float32),
        W_w.astype(jnp.float32),
        U.astype(jnp.float32),
    )

    out, hidden_emd = pl.pallas_call(
        _combine_kernel,
        grid=(2,),
        out_shape=(
            jax.ShapeDtypeStruct((N, out_c), jnp.float32),
            jax.ShapeDtypeStruct((N, hidden), jnp.float32),
        ),
        in_specs=[
            pl.BlockSpec((2, N, hidden), lambda c: (0, 0, 0)),
            pl.BlockSpec((1, hidden), lambda c: (0, 0)),
            pl.BlockSpec((1, hidden), lambda c: (0, 0)),
            pl.BlockSpec((1, hidden), lambda c: (0, 0)),
            pl.BlockSpec((out_c, hidden), lambda c: (0, 0)),
            pl.BlockSpec((1, out_c), lambda c: (0, 0)),
        ],
        out_specs=(
            pl.BlockSpec((nh, out_c), lambda c: (c, 0)),
            pl.BlockSpec((nh, hidden), lambda c: (c, 0)),
        ),
        compiler_params=pltpu.CompilerParams(
            dimension_semantics=("parallel",)),
    )(
        hp,
        W_b.reshape(1, hidden).astype(jnp.float32),
        bn_gamma.reshape(1, hidden).astype(jnp.float32),
        bn_beta.reshape(1, hidden).astype(jnp.float32),
        MLP_w.astype(jnp.float32),
        MLP_b.reshape(1, out_c).astype(jnp.float32),
    )
    return out, hidden_emd
